# Initial kernel scaffold; baseline (speedup 1.0000x reference)
import jax
import jax.numpy as jnp
from jax.experimental import pallas as pl


def kernel(pos):
    raise NotImplementedError("write your pallas kernel here")


# speedup vs baseline: 11.7514x; 11.7514x over previous
import functools

import jax
import jax.numpy as jnp
from jax import lax
from jax.experimental import pallas as pl
from jax.experimental.pallas import tpu as pltpu
from jax.experimental.pallas import tpu_sc as plsc

N = 65536
N_SAMPLES = 4096
NS = 16
L = 16
PER_TILE = N // NS
BLOCKS = PER_TILE // L
INT_MAX = 2**31 - 1


def _fps_sc(xs, ys, zs):
    mesh = plsc.VectorSubcoreMesh(core_axis_name="c", subcore_axis_name="s")

    @functools.partial(
        pl.kernel,
        out_type=jax.ShapeDtypeStruct((N_SAMPLES,), jnp.int32),
        mesh=mesh,
        compiler_params=pltpu.CompilerParams(needs_layout_passes=False),
        scratch_types=[
            pltpu.VMEM((PER_TILE,), jnp.float32),
            pltpu.VMEM((PER_TILE,), jnp.float32),
            pltpu.VMEM((PER_TILE,), jnp.float32),
            pltpu.VMEM((PER_TILE,), jnp.float32),
            pltpu.VMEM((N_SAMPLES,), jnp.int32),
            pltpu.VMEM((L,), jnp.int32),
            pltpu.VMEM((NS * L,), jnp.int32),
            pltpu.VMEM((L,), jnp.int32),
            pltpu.VMEM_SHARED((NS * L,), jnp.int32),
            pltpu.VMEM_SHARED((L,), jnp.int32),
        ],
    )
    def k(xs_hbm, ys_hbm, zs_hbm, out_hbm,
          x_v, y_v, z_v, dist_v, out_v, rec_v, allrec_v, res_v,
          shared_recs, shared_res):
        cid = lax.axis_index("c")
        sid = lax.axis_index("s")
        iota = lax.iota(jnp.int32, L)
        lane0 = iota == 0

        @pl.when(cid == 0)
        def _core0():
            base = sid * PER_TILE
            pltpu.sync_copy(xs_hbm.at[pl.ds(base, PER_TILE)], x_v)
            pltpu.sync_copy(ys_hbm.at[pl.ds(base, PER_TILE)], y_v)
            pltpu.sync_copy(zs_hbm.at[pl.ds(base, PER_TILE)], z_v)

            def init_body(j, _):
                dist_v[pl.ds(j * L, L)] = jnp.full((L,), jnp.inf, jnp.float32)
                return 0
            lax.fori_loop(0, BLOCKS, init_body, 0)

            def put_rec(val_bits, idx, xb, yb, zb):
                rec = jnp.where(iota == 0, val_bits,
                      jnp.where(iota == 1, idx,
                      jnp.where(iota == 2, xb,
                      jnp.where(iota == 3, yb,
                      jnp.where(iota == 4, zb, jnp.int32(0))))))
                rec_v[...] = rec
                return rec

            @pl.when(sid == 0)
            def _prologue():
                x16 = lax.bitcast_convert_type(x_v[pl.ds(0, L)], jnp.int32)
                y16 = lax.bitcast_convert_type(y_v[pl.ds(0, L)], jnp.int32)
                z16 = lax.bitcast_convert_type(z_v[pl.ds(0, L)], jnp.int32)
                xb = jnp.sum(jnp.where(lane0, x16, 0))
                yb = jnp.sum(jnp.where(lane0, y16, 0))
                zb = jnp.sum(jnp.where(lane0, z16, 0))
                res_v[...] = put_rec(jnp.int32(0), jnp.int32(0), xb, yb, zb)
                pltpu.sync_copy(res_v, shared_res)

            plsc.subcore_barrier()

            def outer(i, _):
                pltpu.sync_copy(shared_res, res_v)
                cur_vec = plsc.load_gather(res_v, [jnp.full((L,), 1, jnp.int32)])
                cxv = lax.bitcast_convert_type(
                    plsc.load_gather(res_v, [jnp.full((L,), 2, jnp.int32)]),
                    jnp.float32)
                cyv = lax.bitcast_convert_type(
                    plsc.load_gather(res_v, [jnp.full((L,), 3, jnp.int32)]),
                    jnp.float32)
                czv = lax.bitcast_convert_type(
                    plsc.load_gather(res_v, [jnp.full((L,), 4, jnp.int32)]),
                    jnp.float32)

                @pl.when(sid == 0)
                def _store_idx():
                    plsc.store_scatter(out_v, [jnp.full((L,), i, jnp.int32)],
                                       cur_vec, mask=lane0)

                base_idx = sid * PER_TILE

                def block(j, carry):
                    bv, bi, bxv, byv, bzv = carry
                    sl = pl.ds(j * L, L)
                    xb = x_v[sl]
                    yb = y_v[sl]
                    zb = z_v[sl]
                    dx = xb - cxv
                    dy = yb - cyv
                    dz = zb - czv
                    d = dx * dx + dy * dy
                    d = d + dz * dz
                    nd = jnp.minimum(dist_v[sl], d)
                    dist_v[sl] = nd
                    iv = iota + (base_idx + j * L)
                    cmp = nd > bv
                    bv = jnp.where(cmp, nd, bv)
                    bi = jnp.where(cmp, iv, bi)
                    bxv = jnp.where(cmp, xb, bxv)
                    byv = jnp.where(cmp, yb, byv)
                    bzv = jnp.where(cmp, zb, bzv)
                    return bv, bi, bxv, byv, bzv

                zf = jnp.zeros((L,), jnp.float32)
                zi = jnp.zeros((L,), jnp.int32)
                bv, bi, bxv, byv, bzv = lax.fori_loop(
                    0, BLOCKS, block,
                    (jnp.full((L,), -1.0, jnp.float32), zi, zf, zf, zf))

                m = jnp.max(bv)
                cand = jnp.where(bv == m, bi, INT_MAX)
                li = jnp.min(cand)
                wmask = cand == li
                vb = jnp.sum(jnp.where(wmask, lax.bitcast_convert_type(bv, jnp.int32), 0))
                xb = jnp.sum(jnp.where(wmask, lax.bitcast_convert_type(bxv, jnp.int32), 0))
                yb = jnp.sum(jnp.where(wmask, lax.bitcast_convert_type(byv, jnp.int32), 0))
                zb = jnp.sum(jnp.where(wmask, lax.bitcast_convert_type(bzv, jnp.int32), 0))
                put_rec(vb, li, xb, yb, zb)
                pltpu.sync_copy(rec_v, shared_recs.at[pl.ds(sid * L, L)])

                plsc.subcore_barrier()

                @pl.when(sid == 0)
                def _reduce():
                    pltpu.sync_copy(shared_recs, allrec_v)
                    vals = plsc.load_gather(allrec_v, [iota * L])
                    mm = jnp.max(vals)
                    tc = jnp.where(vals == mm, iota, INT_MAX)
                    t = jnp.min(tc)
                    res_v[...] = allrec_v[pl.ds(t * L, L)]
                    pltpu.sync_copy(res_v, shared_res)

                plsc.subcore_barrier()
                return 0

            lax.fori_loop(0, N_SAMPLES, outer, 0)

            @pl.when(sid == 0)
            def _epilogue():
                pltpu.sync_copy(out_v, out_hbm)

    return k(xs, ys, zs)


def kernel(pos):
    xs = jnp.asarray(pos[:, 0])
    ys = jnp.asarray(pos[:, 1])
    zs = jnp.asarray(pos[:, 2])
    return _fps_sc(xs, ys, zs)


# speedup vs baseline: 13.6676x; 1.1631x over previous
import functools

import jax
import jax.numpy as jnp
from jax import lax
from jax.experimental import pallas as pl
from jax.experimental.pallas import tpu as pltpu
from jax.experimental.pallas import tpu_sc as plsc

N = 65536
N_SAMPLES = 4096
NS = 16
L = 16
PER_TILE = N // NS
BLOCKS = PER_TILE // L
INT_MAX = 2**31 - 1


def _fps_sc(xs, ys, zs):
    mesh = plsc.VectorSubcoreMesh(core_axis_name="c", subcore_axis_name="s")

    @functools.partial(
        pl.kernel,
        out_type=jax.ShapeDtypeStruct((N_SAMPLES,), jnp.int32),
        mesh=mesh,
        compiler_params=pltpu.CompilerParams(needs_layout_passes=False),
        scratch_types=[
            pltpu.VMEM((PER_TILE,), jnp.float32),
            pltpu.VMEM((PER_TILE,), jnp.float32),
            pltpu.VMEM((PER_TILE,), jnp.float32),
            pltpu.VMEM((PER_TILE,), jnp.float32),
            pltpu.VMEM((N_SAMPLES,), jnp.int32),
            pltpu.VMEM((L,), jnp.int32),
            pltpu.VMEM((NS * L,), jnp.int32),
            pltpu.VMEM((L,), jnp.int32),
            pltpu.VMEM_SHARED((NS * L,), jnp.int32),
            pltpu.VMEM_SHARED((L,), jnp.int32),
        ],
    )
    def k(xs_hbm, ys_hbm, zs_hbm, out_hbm,
          x_v, y_v, z_v, dist_v, out_v, rec_v, allrec_v, res_v,
          shared_recs, shared_res):
        cid = lax.axis_index("c")
        sid = lax.axis_index("s")
        iota = lax.iota(jnp.int32, L)
        lane0 = iota == 0

        @pl.when(cid == 0)
        def _core0():
            base = sid * PER_TILE
            pltpu.sync_copy(xs_hbm.at[pl.ds(base, PER_TILE)], x_v)
            pltpu.sync_copy(ys_hbm.at[pl.ds(base, PER_TILE)], y_v)
            pltpu.sync_copy(zs_hbm.at[pl.ds(base, PER_TILE)], z_v)

            def init_body(j, _):
                dist_v[pl.ds(j * L, L)] = jnp.full((L,), jnp.inf, jnp.float32)
                return 0
            lax.fori_loop(0, BLOCKS, init_body, 0)

            def put_rec(val_bits, idx, xb, yb, zb):
                rec = jnp.where(iota == 0, val_bits,
                      jnp.where(iota == 1, idx,
                      jnp.where(iota == 2, xb,
                      jnp.where(iota == 3, yb,
                      jnp.where(iota == 4, zb, jnp.int32(0))))))
                rec_v[...] = rec
                return rec

            @pl.when(sid == 0)
            def _prologue():
                x16 = lax.bitcast_convert_type(x_v[pl.ds(0, L)], jnp.int32)
                y16 = lax.bitcast_convert_type(y_v[pl.ds(0, L)], jnp.int32)
                z16 = lax.bitcast_convert_type(z_v[pl.ds(0, L)], jnp.int32)
                xb = jnp.sum(jnp.where(lane0, x16, 0))
                yb = jnp.sum(jnp.where(lane0, y16, 0))
                zb = jnp.sum(jnp.where(lane0, z16, 0))
                res_v[...] = put_rec(jnp.int32(0), jnp.int32(0), xb, yb, zb)
                pltpu.sync_copy(res_v, shared_res)

            plsc.subcore_barrier()

            def outer(i, _):
                pltpu.sync_copy(shared_res, res_v)
                cur_vec = plsc.load_gather(res_v, [jnp.full((L,), 1, jnp.int32)])
                cxv = lax.bitcast_convert_type(
                    plsc.load_gather(res_v, [jnp.full((L,), 2, jnp.int32)]),
                    jnp.float32)
                cyv = lax.bitcast_convert_type(
                    plsc.load_gather(res_v, [jnp.full((L,), 3, jnp.int32)]),
                    jnp.float32)
                czv = lax.bitcast_convert_type(
                    plsc.load_gather(res_v, [jnp.full((L,), 4, jnp.int32)]),
                    jnp.float32)

                @pl.when(sid == 0)
                def _store_idx():
                    plsc.store_scatter(out_v, [jnp.full((L,), i, jnp.int32)],
                                       cur_vec, mask=lane0)

                base_idx = sid * PER_TILE

                zf = jnp.zeros((L,), jnp.float32)
                zi = jnp.zeros((L,), jnp.int32)

                @plsc.parallel_loop(
                    0, PER_TILE, L, unroll=8,
                    carry=(jnp.full((L,), -1.0, jnp.float32), zi, zf, zf, zf))
                def block(off, carry):
                    bv, bi, bxv, byv, bzv = carry
                    sl = pl.ds(off, L)
                    xb = x_v[sl]
                    yb = y_v[sl]
                    zb = z_v[sl]
                    dx = xb - cxv
                    dy = yb - cyv
                    dz = zb - czv
                    d = dx * dx + dy * dy
                    d = d + dz * dz
                    nd = jnp.minimum(dist_v[sl], d)
                    dist_v[sl] = nd
                    iv = iota + (base_idx + off)
                    cmp = nd > bv
                    bv = jnp.where(cmp, nd, bv)
                    bi = jnp.where(cmp, iv, bi)
                    bxv = jnp.where(cmp, xb, bxv)
                    byv = jnp.where(cmp, yb, byv)
                    bzv = jnp.where(cmp, zb, bzv)
                    return bv, bi, bxv, byv, bzv

                bv, bi, bxv, byv, bzv = block

                m = jnp.max(bv)
                cand = jnp.where(bv == m, bi, INT_MAX)
                li = jnp.min(cand)
                wmask = cand == li
                vb = jnp.sum(jnp.where(wmask, lax.bitcast_convert_type(bv, jnp.int32), 0))
                xb = jnp.sum(jnp.where(wmask, lax.bitcast_convert_type(bxv, jnp.int32), 0))
                yb = jnp.sum(jnp.where(wmask, lax.bitcast_convert_type(byv, jnp.int32), 0))
                zb = jnp.sum(jnp.where(wmask, lax.bitcast_convert_type(bzv, jnp.int32), 0))
                put_rec(vb, li, xb, yb, zb)
                pltpu.sync_copy(rec_v, shared_recs.at[pl.ds(sid * L, L)])

                plsc.subcore_barrier()

                @pl.when(sid == 0)
                def _reduce():
                    pltpu.sync_copy(shared_recs, allrec_v)
                    vals = plsc.load_gather(allrec_v, [iota * L])
                    mm = jnp.max(vals)
                    tc = jnp.where(vals == mm, iota, INT_MAX)
                    t = jnp.min(tc)
                    res_v[...] = allrec_v[pl.ds(t * L, L)]
                    pltpu.sync_copy(res_v, shared_res)

                plsc.subcore_barrier()
                return 0

            lax.fori_loop(0, N_SAMPLES, outer, 0)

            @pl.when(sid == 0)
            def _epilogue():
                pltpu.sync_copy(out_v, out_hbm)

    return k(xs, ys, zs)


def kernel(pos):
    xs = jnp.asarray(pos[:, 0])
    ys = jnp.asarray(pos[:, 1])
    zs = jnp.asarray(pos[:, 2])
    return _fps_sc(xs, ys, zs)


# speedup vs baseline: 15.8025x; 1.1562x over previous
import functools

import jax
import jax.numpy as jnp
from jax import lax
from jax.experimental import pallas as pl
from jax.experimental.pallas import tpu as pltpu
from jax.experimental.pallas import tpu_sc as plsc

N = 65536
N_SAMPLES = 4096
NS = 16
L = 16
PER_TILE = N // NS
INT_MAX = 2**31 - 1


def _fps_sc(xs, ys, zs):
    mesh = plsc.VectorSubcoreMesh(core_axis_name="c", subcore_axis_name="s")

    @functools.partial(
        pl.kernel,
        out_type=jax.ShapeDtypeStruct((N_SAMPLES,), jnp.int32),
        mesh=mesh,
        compiler_params=pltpu.CompilerParams(needs_layout_passes=False),
        scratch_types=[
            pltpu.VMEM((PER_TILE,), jnp.float32),
            pltpu.VMEM((PER_TILE,), jnp.float32),
            pltpu.VMEM((PER_TILE,), jnp.float32),
            pltpu.VMEM((PER_TILE,), jnp.float32),
            pltpu.VMEM((N_SAMPLES,), jnp.int32),
            pltpu.VMEM((L,), jnp.int32),
            pltpu.VMEM((NS * L,), jnp.int32),
            pltpu.VMEM((L,), jnp.int32),
            pltpu.VMEM_SHARED((NS * L,), jnp.int32),
            pltpu.VMEM_SHARED((L,), jnp.int32),
        ],
    )
    def k(xs_hbm, ys_hbm, zs_hbm, out_hbm,
          x_v, y_v, z_v, dist_v, out_v, rec_v, allrec_v, res_v,
          shared_recs, shared_res):
        cid = lax.axis_index("c")
        sid = lax.axis_index("s")
        iota = lax.iota(jnp.int32, L)
        lane0 = iota == 0

        @pl.when(cid == 0)
        def _core0():
            base = sid * PER_TILE
            sl_t = pl.ds(base, PER_TILE)
            pltpu.sync_copy(xs_hbm.at[sl_t], x_v)
            pltpu.sync_copy(ys_hbm.at[sl_t], y_v)
            pltpu.sync_copy(zs_hbm.at[sl_t], z_v)

            @plsc.parallel_loop(0, PER_TILE, L, unroll=8)
            def _init(off):
                dist_v[pl.ds(off, L)] = jnp.full((L,), jnp.inf, jnp.float32)

            def make_rec(val_bits, idx, xb, yb, zb):
                return jnp.where(iota == 0, val_bits,
                       jnp.where(iota == 1, idx,
                       jnp.where(iota == 2, xb,
                       jnp.where(iota == 3, yb,
                       jnp.where(iota == 4, zb, jnp.int32(0))))))

            def coord_bits(ref, loc):
                return lax.bitcast_convert_type(
                    plsc.load_gather(ref, [loc]), jnp.int32)

            @pl.when(sid == 0)
            def _prologue():
                zloc = jnp.zeros((L,), jnp.int32)
                res_v[...] = make_rec(jnp.int32(0), jnp.int32(0),
                                      coord_bits(x_v, zloc),
                                      coord_bits(y_v, zloc),
                                      coord_bits(z_v, zloc))
                pltpu.sync_copy(res_v, shared_res)

            plsc.subcore_barrier()

            def outer(i, _):
                pltpu.sync_copy(shared_res, res_v)
                cur_vec = plsc.load_gather(res_v, [jnp.full((L,), 1, jnp.int32)])
                cxv = lax.bitcast_convert_type(
                    plsc.load_gather(res_v, [jnp.full((L,), 2, jnp.int32)]),
                    jnp.float32)
                cyv = lax.bitcast_convert_type(
                    plsc.load_gather(res_v, [jnp.full((L,), 3, jnp.int32)]),
                    jnp.float32)
                czv = lax.bitcast_convert_type(
                    plsc.load_gather(res_v, [jnp.full((L,), 4, jnp.int32)]),
                    jnp.float32)

                @pl.when(sid == 0)
                def _store_idx():
                    plsc.store_scatter(out_v, [jnp.full((L,), i, jnp.int32)],
                                       cur_vec, mask=lane0)

                base_idx = sid * PER_TILE
                zi = jnp.zeros((L,), jnp.int32)
                neg1 = jnp.full((L,), -1.0, jnp.float32)

                def upd(tr, off):
                    bv, bi = tr
                    sl = pl.ds(off, L)
                    dx = x_v[sl] - cxv
                    dy = y_v[sl] - cyv
                    dz = z_v[sl] - czv
                    d = dx * dx + dy * dy
                    d = d + dz * dz
                    nd = jnp.minimum(dist_v[sl], d)
                    dist_v[sl] = nd
                    iv = iota + (base_idx + off)
                    cmp = nd > bv
                    return jnp.where(cmp, nd, bv), jnp.where(cmp, iv, bi)

                @plsc.parallel_loop(
                    0, PER_TILE, 2 * L, unroll=4,
                    carry=((neg1, zi), (neg1, zi)))
                def block(off, carry):
                    ta, tb = carry
                    return upd(ta, off), upd(tb, off + L)

                ta, tb = block
                take_a = (ta[0] > tb[0]) | ((ta[0] == tb[0]) & (ta[1] < tb[1]))
                bv = jnp.where(take_a, ta[0], tb[0])
                bi = jnp.where(take_a, ta[1], tb[1])

                m = jnp.max(bv)
                cand = jnp.where(bv == m, bi, INT_MAX)
                li = jnp.min(cand)
                wmask = cand == li
                vb = jnp.sum(jnp.where(
                    wmask, lax.bitcast_convert_type(bv, jnp.int32), 0))
                loc = jnp.broadcast_to(li, (L,)) - base_idx
                rec_v[...] = make_rec(vb, li,
                                      coord_bits(x_v, loc),
                                      coord_bits(y_v, loc),
                                      coord_bits(z_v, loc))
                pltpu.sync_copy(rec_v, shared_recs.at[pl.ds(sid * L, L)])

                plsc.subcore_barrier()

                @pl.when(sid == 0)
                def _reduce():
                    pltpu.sync_copy(shared_recs, allrec_v)
                    vals = plsc.load_gather(allrec_v, [iota * L])
                    mm = jnp.max(vals)
                    tc = jnp.where(vals == mm, iota, INT_MAX)
                    t = jnp.min(tc)
                    res_v[...] = allrec_v[pl.ds(t * L, L)]
                    pltpu.sync_copy(res_v, shared_res)

                plsc.subcore_barrier()
                return 0

            lax.fori_loop(0, N_SAMPLES, outer, 0)

            @pl.when(sid == 0)
            def _epilogue():
                pltpu.sync_copy(out_v, out_hbm)

    return k(xs, ys, zs)


def kernel(pos):
    xs = jnp.asarray(pos[:, 0])
    ys = jnp.asarray(pos[:, 1])
    zs = jnp.asarray(pos[:, 2])
    return _fps_sc(xs, ys, zs)
